# Initial kernel scaffold; baseline (speedup 1.0000x reference)
import jax
import jax.numpy as jnp
from jax.experimental import pallas as pl


def kernel(prefix, table):
    raise NotImplementedError("write your pallas kernel here")


# speedup vs baseline: 1.5506x; 1.5506x over previous
import functools

import jax
import jax.numpy as jnp
from jax import lax
from jax.experimental import pallas as pl
from jax.experimental.pallas import tpu as pltpu
from jax.experimental.pallas import tpu_sc as plsc

PRE_SEQ_LEN = 128
HIDDEN = 768
ROW_DIM = 12 * 2 * HIDDEN
BATCH = 16
PREFIX_LEN = 128
B = BATCH * PREFIX_LEN

NC, NS = 2, 16
NW = NC * NS
B_PER_W = B // NW
CHUNK = 4
NCHUNKS = B_PER_W // CHUNK


def _make_kernel():
    mesh = plsc.VectorSubcoreMesh(core_axis_name="c", subcore_axis_name="s")

    @functools.partial(
        pl.kernel,
        mesh=mesh,
        out_type=jax.ShapeDtypeStruct((B, ROW_DIM), jnp.float32),
        scratch_types=[
            pltpu.VMEM((NCHUNKS, CHUNK), jnp.int32),
            pltpu.VMEM((CHUNK, ROW_DIM), jnp.float32),
            pltpu.SemaphoreType.DMA,
            pltpu.SemaphoreType.DMA,
        ],
    )
    def gather_kernel(table_hbm, idx_hbm, out_hbm, idx_v, rows_v, gsem, ssem):
        wid = lax.axis_index("s") * NC + lax.axis_index("c")
        base = wid * B_PER_W
        pltpu.sync_copy(idx_hbm.at[wid], idx_v)

        def body(c, carry):
            gather = pltpu.make_async_copy(
                table_hbm.at[idx_v.at[c]], rows_v, gsem)
            gather.start()
            gather.wait()
            scatter = pltpu.make_async_copy(
                rows_v, out_hbm.at[pl.ds(base + c * CHUNK, CHUNK)], ssem)
            scatter.start()
            scatter.wait()
            return carry

        lax.fori_loop(0, NCHUNKS, body, 0)

    return gather_kernel


_GATHER = _make_kernel()


@jax.jit
def kernel(prefix, table):
    idx = prefix.reshape(NW, NCHUNKS, CHUNK).astype(jnp.int32)
    out = _GATHER(table, idx)
    return out.reshape(BATCH, PREFIX_LEN, ROW_DIM)


# speedup vs baseline: 1.6569x; 1.0686x over previous
import functools

import jax
import jax.numpy as jnp
from jax import lax
from jax.experimental import pallas as pl
from jax.experimental.pallas import tpu as pltpu
from jax.experimental.pallas import tpu_sc as plsc

PRE_SEQ_LEN = 128
HIDDEN = 768
ROW_DIM = 12 * 2 * HIDDEN
BATCH = 16
PREFIX_LEN = 128
B = BATCH * PREFIX_LEN

NC, NS = 2, 16
NW = NC * NS
B_PER_W = B // NW
CHUNK = 2
NBUF = 2
NCHUNKS = B_PER_W // CHUNK
NGROUPS = NCHUNKS // NBUF


def _make_kernel():
    mesh = plsc.VectorSubcoreMesh(core_axis_name="c", subcore_axis_name="s")

    @functools.partial(
        pl.kernel,
        mesh=mesh,
        out_type=jax.ShapeDtypeStruct((B, ROW_DIM), jnp.float32),
        scratch_types=[
            pltpu.VMEM((NCHUNKS, CHUNK), jnp.int32),
            pltpu.VMEM((NBUF, CHUNK, ROW_DIM), jnp.float32),
            pltpu.SemaphoreType.DMA,
            pltpu.SemaphoreType.DMA,
            pltpu.SemaphoreType.DMA,
            pltpu.SemaphoreType.DMA,
        ],
    )
    def gather_kernel(table_hbm, idx_hbm, out_hbm, idx_v, rows_v,
                      gsem0, gsem1, ssem0, ssem1):
        gsems = (gsem0, gsem1)
        ssems = (ssem0, ssem1)
        wid = lax.axis_index("s") * NC + lax.axis_index("c")
        base = wid * B_PER_W
        pltpu.sync_copy(idx_hbm.at[wid], idx_v)

        for b in range(NBUF):
            pltpu.async_copy(table_hbm.at[idx_v.at[b]], rows_v.at[b],
                             gsems[b])

        def body(g, carry):
            for b in range(NBUF):
                c = g * NBUF + b
                pltpu.make_async_copy(
                    table_hbm.at[idx_v.at[c]], rows_v.at[b],
                    gsems[b]).wait()
                pltpu.async_copy(
                    rows_v.at[b],
                    out_hbm.at[pl.ds(base + c * CHUNK, CHUNK)], ssems[b])

                @pl.when(c + NBUF < NCHUNKS)
                def _():
                    pltpu.make_async_copy(
                        rows_v.at[b],
                        out_hbm.at[pl.ds(base, CHUNK)], ssems[b]).wait()
                    pltpu.async_copy(
                        table_hbm.at[idx_v.at[c + NBUF]], rows_v.at[b],
                        gsems[b])
            return carry

        lax.fori_loop(0, NGROUPS, body, 0)
        for b in range(NBUF):
            pltpu.make_async_copy(
                rows_v.at[b],
                out_hbm.at[pl.ds(base, CHUNK)], ssems[b]).wait()

    return gather_kernel


_GATHER = _make_kernel()


@jax.jit
def kernel(prefix, table):
    idx = prefix.reshape(NW, NCHUNKS, CHUNK).astype(jnp.int32)
    out = _GATHER(table, idx)
    return out.reshape(BATCH, PREFIX_LEN, ROW_DIM)
